# Initial kernel scaffold; baseline (speedup 1.0000x reference)
import functools

import jax
import jax.numpy as jnp
from jax import lax
from jax.experimental import pallas as pl
from jax.experimental.pallas import tpu as pltpu
from jax.experimental.pallas import tpu_sc as plsc

N = 10000
D = 128
G = 64
NC = 2
NS = 16
NT = NC * NS
CH = 128
NCH = 80
EPT = CH * NCH
RPT = N // NS

_MESH = dict(mesh=plsc.VectorSubcoreMesh(core_axis_name="c", subcore_axis_name="s"))


def _zero_buf(buf, rows, width):
    def body(j, _):
        for fg in range(width // 16):
            buf[j, pl.ds(fg * 16, 16)] = jnp.zeros((16,), jnp.float32)
        return 0
    lax.fori_loop(0, rows, body, 0)


def _zero_shared_slice(src_buf, shared, base):
    nfull = RPT // CH
    rem = RPT - nfull * CH
    for k in range(nfull):
        pltpu.sync_copy(src_buf, shared.at[pl.ds(base + k * CH, CH)])
    if rem:
        pltpu.sync_copy(src_buf.at[pl.ds(0, rem)],
                        shared.at[pl.ds(base + nfull * CH, rem)])


@functools.partial(
    pl.kernel,
    out_type=jax.ShapeDtypeStruct((NC, N, 16), jnp.float32),
    scratch_types=[
        pltpu.VMEM((NCH, CH), jnp.int32),
        pltpu.VMEM((NCH, CH), jnp.float32),
        pltpu.VMEM((CH, 16), jnp.float32),
        pltpu.VMEM_SHARED((N, 16), jnp.float32),
    ],
    **_MESH,
)
def _deg_kernel(col_hbm, w_hbm, out_hbm, cols_v, ws_v, dbuf, deg_sh):
    cid = lax.axis_index("c")
    sid = lax.axis_index("s")
    t = cid * NS + sid
    pltpu.sync_copy(col_hbm.at[t], cols_v)
    pltpu.sync_copy(w_hbm.at[t], ws_v)
    _zero_buf(dbuf, CH, 16)
    _zero_shared_slice(dbuf, deg_sh, sid * RPT)
    plsc.subcore_barrier()

    def chunk(g, _):
        def fill(j, _):
            dbuf[j, :] = jnp.broadcast_to(ws_v[g, j], (16,))
            return 0
        lax.fori_loop(0, CH, fill, 0)
        pltpu.sync_copy(dbuf, deg_sh.at[cols_v.at[g]], add=True)
        return 0

    lax.fori_loop(0, NCH, chunk, 0)
    plsc.subcore_barrier()
    pltpu.sync_copy(deg_sh.at[pl.ds(sid * RPT, RPT)],
                    out_hbm.at[cid, pl.ds(sid * RPT, RPT)])


@functools.partial(
    pl.kernel,
    out_type=jax.ShapeDtypeStruct((NC, N, D), jnp.float32),
    scratch_types=[
        pltpu.VMEM((NCH, CH), jnp.int32),
        pltpu.VMEM((NCH, CH), jnp.int32),
        pltpu.VMEM((NCH, CH), jnp.float32),
        pltpu.VMEM((CH, D), jnp.float32),
        pltpu.VMEM_SHARED((N, D), jnp.float32),
        pltpu.SemaphoreType.DMA,
    ],
    **_MESH,
)
def _scatter_kernel(y_hbm, row_hbm, col_hbm, w_hbm, out_hbm,
                    rows_v, cols_v, ws_v, buf, acc_sh, sem):
    cid = lax.axis_index("c")
    sid = lax.axis_index("s")
    t = cid * NS + sid
    pltpu.sync_copy(row_hbm.at[t], rows_v)
    pltpu.sync_copy(col_hbm.at[t], cols_v)
    pltpu.sync_copy(w_hbm.at[t], ws_v)
    _zero_buf(buf, CH, D)
    _zero_shared_slice(buf, acc_sh, sid * RPT)
    plsc.subcore_barrier()

    def chunk(g, _):
        pltpu.async_copy(y_hbm.at[rows_v.at[g]], buf, sem).wait()

        def scale(j, _):
            w = ws_v[g, j]
            for fg in range(D // 16):
                sl = pl.ds(fg * 16, 16)
                buf[j, sl] = buf[j, sl] * w
            return 0

        lax.fori_loop(0, CH, scale, 0)
        pltpu.sync_copy(buf, acc_sh.at[cols_v.at[g]], add=True)
        return 0

    lax.fori_loop(0, NCH, chunk, 0)
    plsc.subcore_barrier()
    pltpu.sync_copy(acc_sh.at[pl.ds(sid * RPT, RPT)],
                    out_hbm.at[cid, pl.ds(sid * RPT, RPT)])


BLK = 1000
GRID = N // BLK


def _dinv_from(degp):
    deg = degp[0, :, 0] + degp[1, :, 0] + 1.0
    pos = deg > 0
    return jnp.where(pos, 1.0, 0.0) / jnp.sqrt(jnp.where(pos, deg, 1.0))


def _leaky(v):
    return jnp.where(v >= 0, v, 0.01 * v)


def _tc_first_body(x_ref, w1_ref, degp_ref, y_ref, aux_ref):
    dinv = _dinv_from(degp_ref[...])
    xw = jnp.dot(x_ref[...], w1_ref[...], preferred_element_type=jnp.float32)
    y_ref[...] = xw * dinv[:, None]
    aux_ref[...] = xw * (dinv * dinv)[:, None]


def _tc_mid_body(sp_ref, aux_ref, degp_ref, b1_ref, w2_ref, y_ref, aux2_ref):
    dinv = _dinv_from(degp_ref[...])
    sp = sp_ref[...]
    h = _leaky((sp[0] + sp[1]) * dinv[:, None] + aux_ref[...] + b1_ref[...])
    xw = jnp.dot(h, w2_ref[...], preferred_element_type=jnp.float32)
    y_ref[...] = xw * dinv[:, None]
    aux2_ref[...] = xw * (dinv * dinv)[:, None]


def _tc_head_body(sp_ref, aux_ref, degp_ref, b2_ref, batch_ref,
                  wf1_ref, bf1_ref, wf2_ref, bf2_ref, wf3_ref, bf3_ref,
                  out_ref, psum, pcnt):
    i = pl.program_id(0)

    @pl.when(i == 0)
    def _():
        psum[...] = jnp.zeros_like(psum)
        pcnt[...] = jnp.zeros_like(pcnt)

    dinv = _dinv_from(degp_ref[...])
    sp = sp_ref[...]
    h = (sp[0] + sp[1]) * dinv[:, None] + aux_ref[...] + b2_ref[...]
    bidx = batch_ref[0, 0, :]
    onehot = (bidx[None, :] == lax.broadcasted_iota(jnp.int32, (G, BLK), 0)
              ).astype(jnp.float32)
    psum[...] += jnp.dot(onehot, h, preferred_element_type=jnp.float32)
    pcnt[...] += jnp.broadcast_to(jnp.sum(onehot, axis=1)[:, None], (G, D))

    @pl.when(i == GRID - 1)
    def _():
        pooled = psum[...] / jnp.maximum(pcnt[...], 1.0)
        o = _leaky(jnp.dot(pooled, wf1_ref[...],
                           preferred_element_type=jnp.float32) + bf1_ref[...])
        o = _leaky(jnp.dot(o, wf2_ref[...],
                           preferred_element_type=jnp.float32) + bf2_ref[...])
        out_ref[...] = (jnp.dot(o, wf3_ref[...],
                                preferred_element_type=jnp.float32) + bf3_ref[...])


def _row_spec(last):
    return pl.BlockSpec((BLK, last), lambda i: (i, 0))


def _degp_spec():
    return pl.BlockSpec((2, BLK, 16), lambda i: (0, i, 0))


def _full(shape):
    return pl.BlockSpec(shape, lambda i: tuple(0 for _ in shape))


def _tc_first(x, W1, degp):
    return pl.pallas_call(
        _tc_first_body,
        grid=(GRID,),
        in_specs=[_row_spec(D), _full((D, D)), _degp_spec()],
        out_specs=[_row_spec(D), _row_spec(D)],
        out_shape=[jax.ShapeDtypeStruct((N, D), jnp.float32)] * 2,
    )(x, W1, degp)


def _tc_mid(sp, aux, degp, b1, W2):
    return pl.pallas_call(
        _tc_mid_body,
        grid=(GRID,),
        in_specs=[pl.BlockSpec((2, BLK, D), lambda i: (0, i, 0)),
                  _row_spec(D), _degp_spec(), _full((1, D)), _full((D, D))],
        out_specs=[_row_spec(D), _row_spec(D)],
        out_shape=[jax.ShapeDtypeStruct((N, D), jnp.float32)] * 2,
    )(sp, aux, degp, b1, W2)


def _tc_head(sp, aux, degp, b2, batch3, Wf1, bf1, Wf2, bf2, Wf3, bf3):
    return pl.pallas_call(
        _tc_head_body,
        grid=(GRID,),
        in_specs=[pl.BlockSpec((2, BLK, D), lambda i: (0, i, 0)),
                  _row_spec(D), _degp_spec(), _full((1, D)),
                  pl.BlockSpec((1, 1, BLK), lambda i: (i, 0, 0)),
                  _full((D, D)), _full((1, D)),
                  _full((D, D)), _full((1, D)),
                  _full((D, D)), _full((1, D))],
        out_specs=pl.BlockSpec((G, D), lambda i: (0, 0)),
        out_shape=jax.ShapeDtypeStruct((G, D), jnp.float32),
        scratch_shapes=[pltpu.VMEM((G, D), jnp.float32),
                        pltpu.VMEM((G, D), jnp.float32)],
    )(sp, aux, degp, b2, batch3, Wf1, bf1, Wf2, bf2, Wf3, bf3)


def _pad2d(w, rows, cols):
    return jnp.pad(w, ((0, rows - w.shape[0]), (0, cols - w.shape[1])))


def kernel(x, edge_index, edge_weight, batch, W1, b1, W2, b2,
           Wf1, bf1, Wf2, bf2, Wf3, bf3):
    E = edge_weight.shape[0]
    pad = NT * EPT - E
    row3 = jnp.pad(edge_index[0], (0, pad)).reshape(NT, NCH, CH)
    col3 = jnp.pad(edge_index[1], (0, pad)).reshape(NT, NCH, CH)
    w3 = jnp.pad(edge_weight, (0, pad)).reshape(NT, NCH, CH)
    batch3 = batch.reshape(GRID, 1, BLK)

    degp = _deg_kernel(col3, w3)

    y1, aux1 = _tc_first(x, W1, degp)
    s1 = _scatter_kernel(y1, row3, col3, w3)
    y2, aux2 = _tc_mid(s1, aux1, degp, b1.reshape(1, D), W2)
    s2 = _scatter_kernel(y2, row3, col3, w3)

    out = _tc_head(s2, aux2, degp, b2.reshape(1, D), batch3,
                   _pad2d(Wf1, D, D),
                   jnp.pad(bf1, (0, D - bf1.shape[0])).reshape(1, D),
                   _pad2d(Wf2, D, D),
                   jnp.pad(bf2, (0, D - bf2.shape[0])).reshape(1, D),
                   _pad2d(Wf3, D, D),
                   jnp.pad(bf3, (0, D - bf3.shape[0])).reshape(1, D))
    return out[:, :10]


# speedup vs baseline: 6.5883x; 6.5883x over previous
import functools

import jax
import jax.numpy as jnp
from jax import lax
from jax.experimental import pallas as pl
from jax.experimental.pallas import tpu as pltpu
from jax.experimental.pallas import tpu_sc as plsc

N = 10000
D = 128
G = 64
NC = 2
NS = 16
NT = NC * NS
CH = 128
NCH = 80
EPT = CH * NCH
NP = 10240
RPT = NP // NS

_MESH = dict(mesh=plsc.VectorSubcoreMesh(core_axis_name="c", subcore_axis_name="s"))


def _zero_buf(buf, rows, width):
    def body(j, _):
        for fg in range(width // 16):
            buf[j, pl.ds(fg * 16, 16)] = jnp.zeros((16,), jnp.float32)
        return 0
    lax.fori_loop(0, rows, body, 0)


def _zero_shared_slice(src_buf, shared, base):
    for k in range(RPT // CH):
        pltpu.sync_copy(src_buf, shared.at[pl.ds(base + k * CH, CH)])


@functools.partial(
    pl.kernel,
    out_type=jax.ShapeDtypeStruct((NC, NP, D), jnp.float32),
    scratch_types=[
        pltpu.VMEM((NCH, CH), jnp.int32),
        pltpu.VMEM((NCH, CH), jnp.float32),
        pltpu.VMEM((CH, D), jnp.float32),
        pltpu.VMEM_SHARED((NP, D), jnp.float32),
    ],
    **_MESH,
)
def _deg_kernel(col_hbm, w_hbm, out_hbm, cols_v, ws_v, dbuf, deg_sh):
    cid = lax.axis_index("c")
    sid = lax.axis_index("s")
    t = cid * NS + sid
    pltpu.sync_copy(col_hbm.at[t], cols_v)
    pltpu.sync_copy(w_hbm.at[t], ws_v)
    _zero_buf(dbuf, CH, D)
    _zero_shared_slice(dbuf, deg_sh, sid * RPT)
    plsc.subcore_barrier()

    def chunk(g, _):
        def fill(jj, _):
            wvec = ws_v[g, pl.ds(jj * 16, 16)]
            for l in range(16):
                bw = jnp.broadcast_to(wvec[l], (16,))
                for fg in range(D // 16):
                    dbuf[jj * 16 + l, pl.ds(fg * 16, 16)] = bw
            return 0
        lax.fori_loop(0, CH // 16, fill, 0)
        pltpu.sync_copy(dbuf, deg_sh.at[cols_v.at[g]], add=True)
        return 0

    lax.fori_loop(0, NCH, chunk, 0)
    plsc.subcore_barrier()
    pltpu.sync_copy(deg_sh.at[pl.ds(sid * RPT, RPT)],
                    out_hbm.at[cid, pl.ds(sid * RPT, RPT)])


@functools.partial(
    pl.kernel,
    out_type=jax.ShapeDtypeStruct((NC, NP, D), jnp.float32),
    scratch_types=[
        pltpu.VMEM((NCH, CH), jnp.int32),
        pltpu.VMEM((NCH, CH), jnp.int32),
        pltpu.VMEM((NCH, CH), jnp.float32),
        pltpu.VMEM((CH, D), jnp.float32),
        pltpu.VMEM_SHARED((NP, D), jnp.float32),
        pltpu.SemaphoreType.DMA,
    ],
    **_MESH,
)
def _scatter_kernel(y_hbm, row_hbm, col_hbm, w_hbm, out_hbm,
                    rows_v, cols_v, ws_v, buf, acc_sh, sem):
    cid = lax.axis_index("c")
    sid = lax.axis_index("s")
    t = cid * NS + sid
    pltpu.sync_copy(row_hbm.at[t], rows_v)
    pltpu.sync_copy(col_hbm.at[t], cols_v)
    pltpu.sync_copy(w_hbm.at[t], ws_v)
    _zero_buf(buf, CH, D)
    _zero_shared_slice(buf, acc_sh, sid * RPT)
    plsc.subcore_barrier()

    def chunk(g, _):
        pltpu.async_copy(y_hbm.at[rows_v.at[g]], buf, sem).wait()

        def scale(jj, _):
            wvec = ws_v[g, pl.ds(jj * 16, 16)]
            for l in range(16):
                w = wvec[l]
                j = jj * 16 + l
                for fg in range(D // 16):
                    sl = pl.ds(fg * 16, 16)
                    buf[j, sl] = buf[j, sl] * w
            return 0

        lax.fori_loop(0, CH // 16, scale, 0)
        pltpu.sync_copy(buf, acc_sh.at[cols_v.at[g]], add=True)
        return 0

    lax.fori_loop(0, NCH, chunk, 0)
    plsc.subcore_barrier()
    pltpu.sync_copy(acc_sh.at[pl.ds(sid * RPT, RPT)],
                    out_hbm.at[cid, pl.ds(sid * RPT, RPT)])


BLK = 1000
GRID = N // BLK


def _dinv_from(degp):
    deg = degp[0, :, 0] + degp[1, :, 0] + 1.0
    pos = deg > 0
    return jnp.where(pos, 1.0, 0.0) / jnp.sqrt(jnp.where(pos, deg, 1.0))


def _leaky(v):
    return jnp.where(v >= 0, v, 0.01 * v)


def _tc_first_body(x_ref, w1_ref, degp_ref, y_ref, aux_ref):
    dinv = _dinv_from(degp_ref[...])
    xw = jnp.dot(x_ref[...], w1_ref[...], preferred_element_type=jnp.float32)
    y_ref[...] = xw * dinv[:, None]
    aux_ref[...] = xw * (dinv * dinv)[:, None]


def _tc_mid_body(sp_ref, aux_ref, degp_ref, b1_ref, w2_ref, y_ref, aux2_ref):
    dinv = _dinv_from(degp_ref[...])
    sp = sp_ref[...]
    h = _leaky((sp[0] + sp[1]) * dinv[:, None] + aux_ref[...] + b1_ref[...])
    xw = jnp.dot(h, w2_ref[...], preferred_element_type=jnp.float32)
    y_ref[...] = xw * dinv[:, None]
    aux2_ref[...] = xw * (dinv * dinv)[:, None]


def _tc_head_body(sp_ref, aux_ref, degp_ref, b2_ref, batch_ref,
                  wf1_ref, bf1_ref, wf2_ref, bf2_ref, wf3_ref, bf3_ref,
                  out_ref, psum, pcnt):
    i = pl.program_id(0)

    @pl.when(i == 0)
    def _():
        psum[...] = jnp.zeros_like(psum)
        pcnt[...] = jnp.zeros_like(pcnt)

    dinv = _dinv_from(degp_ref[...])
    sp = sp_ref[...]
    h = (sp[0] + sp[1]) * dinv[:, None] + aux_ref[...] + b2_ref[...]
    bidx = batch_ref[0, 0, :]
    onehot = (bidx[None, :] == lax.broadcasted_iota(jnp.int32, (G, BLK), 0)
              ).astype(jnp.float32)
    psum[...] += jnp.dot(onehot, h, preferred_element_type=jnp.float32)
    pcnt[...] += jnp.broadcast_to(jnp.sum(onehot, axis=1)[:, None], (G, D))

    @pl.when(i == GRID - 1)
    def _():
        pooled = psum[...] / jnp.maximum(pcnt[...], 1.0)
        o = _leaky(jnp.dot(pooled, wf1_ref[...],
                           preferred_element_type=jnp.float32) + bf1_ref[...])
        o = _leaky(jnp.dot(o, wf2_ref[...],
                           preferred_element_type=jnp.float32) + bf2_ref[...])
        out_ref[...] = (jnp.dot(o, wf3_ref[...],
                                preferred_element_type=jnp.float32) + bf3_ref[...])


def _row_spec(last):
    return pl.BlockSpec((BLK, last), lambda i: (i, 0))


def _degp_spec():
    return pl.BlockSpec((2, BLK, D), lambda i: (0, i, 0))


def _full(shape):
    return pl.BlockSpec(shape, lambda i: tuple(0 for _ in shape))


def _tc_first(x, W1, degp):
    return pl.pallas_call(
        _tc_first_body,
        grid=(GRID,),
        in_specs=[_row_spec(D), _full((D, D)), _degp_spec()],
        out_specs=[_row_spec(D), _row_spec(D)],
        out_shape=[jax.ShapeDtypeStruct((N, D), jnp.float32)] * 2,
    )(x, W1, degp)


def _tc_mid(sp, aux, degp, b1, W2):
    return pl.pallas_call(
        _tc_mid_body,
        grid=(GRID,),
        in_specs=[pl.BlockSpec((2, BLK, D), lambda i: (0, i, 0)),
                  _row_spec(D), _degp_spec(), _full((1, D)), _full((D, D))],
        out_specs=[_row_spec(D), _row_spec(D)],
        out_shape=[jax.ShapeDtypeStruct((N, D), jnp.float32)] * 2,
    )(sp, aux, degp, b1, W2)


def _tc_head(sp, aux, degp, b2, batch3, Wf1, bf1, Wf2, bf2, Wf3, bf3):
    return pl.pallas_call(
        _tc_head_body,
        grid=(GRID,),
        in_specs=[pl.BlockSpec((2, BLK, D), lambda i: (0, i, 0)),
                  _row_spec(D), _degp_spec(), _full((1, D)),
                  pl.BlockSpec((1, 1, BLK), lambda i: (i, 0, 0)),
                  _full((D, D)), _full((1, D)),
                  _full((D, D)), _full((1, D)),
                  _full((D, D)), _full((1, D))],
        out_specs=pl.BlockSpec((G, D), lambda i: (0, 0)),
        out_shape=jax.ShapeDtypeStruct((G, D), jnp.float32),
        scratch_shapes=[pltpu.VMEM((G, D), jnp.float32),
                        pltpu.VMEM((G, D), jnp.float32)],
    )(sp, aux, degp, b2, batch3, Wf1, bf1, Wf2, bf2, Wf3, bf3)


def _pad2d(w, rows, cols):
    return jnp.pad(w, ((0, rows - w.shape[0]), (0, cols - w.shape[1])))


def kernel(x, edge_index, edge_weight, batch, W1, b1, W2, b2,
           Wf1, bf1, Wf2, bf2, Wf3, bf3):
    E = edge_weight.shape[0]
    pad = NT * EPT - E
    row3 = jnp.pad(edge_index[0], (0, pad)).reshape(NT, NCH, CH)
    col3 = jnp.pad(edge_index[1], (0, pad)).reshape(NT, NCH, CH)
    w3 = jnp.pad(edge_weight, (0, pad)).reshape(NT, NCH, CH)
    batch3 = batch.reshape(GRID, 1, BLK)

    degp = _deg_kernel(col3, w3)

    y1, aux1 = _tc_first(x, W1, degp)
    s1 = _scatter_kernel(y1, row3, col3, w3)
    y2, aux2 = _tc_mid(s1, aux1, degp, b1.reshape(1, D), W2)
    s2 = _scatter_kernel(y2, row3, col3, w3)

    out = _tc_head(s2, aux2, degp, b2.reshape(1, D), batch3,
                   _pad2d(Wf1, D, D),
                   jnp.pad(bf1, (0, D - bf1.shape[0])).reshape(1, D),
                   _pad2d(Wf2, D, D),
                   jnp.pad(bf2, (0, D - bf2.shape[0])).reshape(1, D),
                   _pad2d(Wf3, D, D),
                   jnp.pad(bf3, (0, D - bf3.shape[0])).reshape(1, D))
    return out[:, :10]


# speedup vs baseline: 6.5895x; 1.0002x over previous
import functools

import jax
import jax.numpy as jnp
from jax import lax
from jax.experimental import pallas as pl
from jax.experimental.pallas import tpu as pltpu
from jax.experimental.pallas import tpu_sc as plsc

N = 10000
D = 128
G = 64
NC = 2
NS = 16
NT = NC * NS
CH = 128
NCH = 80
EPT = CH * NCH
NP = 10240
RPT = NP // NS

_MESH = dict(mesh=plsc.VectorSubcoreMesh(core_axis_name="c", subcore_axis_name="s"))


def _zero_buf(buf, rows, width):
    def body(j, _):
        for fg in range(width // 16):
            buf[j, pl.ds(fg * 16, 16)] = jnp.zeros((16,), jnp.float32)
        return 0
    lax.fori_loop(0, rows, body, 0)


def _zero_shared_slice(src_buf, shared, base, blk):
    for k in range(RPT // blk):
        pltpu.sync_copy(src_buf, shared.at[pl.ds(base + k * blk, blk)])


@functools.partial(
    pl.kernel,
    out_type=jax.ShapeDtypeStruct((NC, NP, D), jnp.float32),
    scratch_types=[
        pltpu.VMEM((NCH, CH), jnp.int32),
        pltpu.VMEM((NCH, CH), jnp.float32),
        pltpu.VMEM((CH, D), jnp.float32),
        pltpu.VMEM_SHARED((NP, D), jnp.float32),
    ],
    **_MESH,
)
def _deg_kernel(col_hbm, w_hbm, out_hbm, cols_v, ws_v, dbuf, deg_sh):
    cid = lax.axis_index("c")
    sid = lax.axis_index("s")
    t = cid * NS + sid
    pltpu.sync_copy(col_hbm.at[t], cols_v)
    pltpu.sync_copy(w_hbm.at[t], ws_v)
    _zero_buf(dbuf, CH, D)
    _zero_shared_slice(dbuf, deg_sh, sid * RPT, CH)
    plsc.subcore_barrier()

    def chunk(g, _):
        def fill(jj, _):
            wvec = ws_v[g, pl.ds(jj * 16, 16)]
            for l in range(16):
                bw = jnp.broadcast_to(wvec[l], (16,))
                for fg in range(D // 16):
                    dbuf[jj * 16 + l, pl.ds(fg * 16, 16)] = bw
            return 0
        lax.fori_loop(0, CH // 16, fill, 0)
        pltpu.sync_copy(dbuf, deg_sh.at[cols_v.at[g]], add=True)
        return 0

    lax.fori_loop(0, NCH, chunk, 0)
    plsc.subcore_barrier()
    pltpu.sync_copy(deg_sh.at[pl.ds(sid * RPT, RPT)],
                    out_hbm.at[cid, pl.ds(sid * RPT, RPT)])


@functools.partial(
    pl.kernel,
    out_type=jax.ShapeDtypeStruct((NC, NP, D), jnp.float32),
    scratch_types=[
        pltpu.VMEM((NCH, CH), jnp.int32),
        pltpu.VMEM((NCH, CH), jnp.int32),
        pltpu.VMEM((NCH, CH), jnp.float32),
        pltpu.VMEM((CH, D), jnp.float32),
        pltpu.VMEM_SHARED((NP, D), jnp.float32),
        pltpu.SemaphoreType.DMA,
    ],
    **_MESH,
)
def _scatter_kernel(y_hbm, row_hbm, col_hbm, w_hbm, out_hbm,
                    rows_v, cols_v, ws_v, buf, acc_sh, sem):
    cid = lax.axis_index("c")
    sid = lax.axis_index("s")
    t = cid * NS + sid
    pltpu.sync_copy(row_hbm.at[t], rows_v)
    pltpu.sync_copy(col_hbm.at[t], cols_v)
    pltpu.sync_copy(w_hbm.at[t], ws_v)
    _zero_buf(buf, CH, D)
    _zero_shared_slice(buf, acc_sh, sid * RPT, CH)
    plsc.subcore_barrier()

    def chunk(g, _):
        pltpu.async_copy(y_hbm.at[rows_v.at[g]], buf, sem).wait()

        def scale(jj, _):
            wvec = ws_v[g, pl.ds(jj * 16, 16)]
            for l in range(16):
                w = wvec[l]
                j = jj * 16 + l
                for fg in range(D // 16):
                    sl = pl.ds(fg * 16, 16)
                    buf[j, sl] = buf[j, sl] * w
            return 0

        lax.fori_loop(0, CH // 16, scale, 0)
        pltpu.sync_copy(buf, acc_sh.at[cols_v.at[g]], add=True)
        return 0

    lax.fori_loop(0, NCH, chunk, 0)
    plsc.subcore_barrier()
    pltpu.sync_copy(acc_sh.at[pl.ds(sid * RPT, RPT)],
                    out_hbm.at[cid, pl.ds(sid * RPT, RPT)])


BLK = 1000
GRID = N // BLK


def _dinv_from(degp):
    deg = degp[0, :, 0] + degp[1, :, 0] + 1.0
    pos = deg > 0
    return jnp.where(pos, 1.0, 0.0) / jnp.sqrt(jnp.where(pos, deg, 1.0))


def _leaky(v):
    return jnp.where(v >= 0, v, 0.01 * v)


def _tc_first_body(x_ref, w1_ref, degp_ref, y_ref, aux_ref):
    dinv = _dinv_from(degp_ref[...])
    xw = jnp.dot(x_ref[...], w1_ref[...], preferred_element_type=jnp.float32)
    y_ref[...] = xw * dinv[:, None]
    aux_ref[...] = xw * (dinv * dinv)[:, None]


def _tc_mid_body(sp_ref, aux_ref, degp_ref, b1_ref, w2_ref, y_ref, aux2_ref):
    dinv = _dinv_from(degp_ref[...])
    sp = sp_ref[...]
    h = _leaky((sp[0] + sp[1]) * dinv[:, None] + aux_ref[...] + b1_ref[...])
    xw = jnp.dot(h, w2_ref[...], preferred_element_type=jnp.float32)
    y_ref[...] = xw * dinv[:, None]
    aux2_ref[...] = xw * (dinv * dinv)[:, None]


def _tc_head_body(sp_ref, aux_ref, degp_ref, b2_ref, batch_ref,
                  wf1_ref, bf1_ref, wf2_ref, bf2_ref, wf3_ref, bf3_ref,
                  out_ref, psum, pcnt):
    i = pl.program_id(0)

    @pl.when(i == 0)
    def _():
        psum[...] = jnp.zeros_like(psum)
        pcnt[...] = jnp.zeros_like(pcnt)

    dinv = _dinv_from(degp_ref[...])
    sp = sp_ref[...]
    h = (sp[0] + sp[1]) * dinv[:, None] + aux_ref[...] + b2_ref[...]
    bidx = batch_ref[0, 0, :]
    onehot = (bidx[None, :] == lax.broadcasted_iota(jnp.int32, (G, BLK), 0)
              ).astype(jnp.float32)
    psum[...] += jnp.dot(onehot, h, preferred_element_type=jnp.float32)
    pcnt[...] += jnp.broadcast_to(jnp.sum(onehot, axis=1)[:, None], (G, D))

    @pl.when(i == GRID - 1)
    def _():
        pooled = psum[...] / jnp.maximum(pcnt[...], 1.0)
        o = _leaky(jnp.dot(pooled, wf1_ref[...],
                           preferred_element_type=jnp.float32) + bf1_ref[...])
        o = _leaky(jnp.dot(o, wf2_ref[...],
                           preferred_element_type=jnp.float32) + bf2_ref[...])
        out_ref[...] = (jnp.dot(o, wf3_ref[...],
                                preferred_element_type=jnp.float32) + bf3_ref[...])


def _row_spec(last):
    return pl.BlockSpec((BLK, last), lambda i: (i, 0))


def _degp_spec():
    return pl.BlockSpec((2, BLK, D), lambda i: (0, i, 0))


def _full(shape):
    return pl.BlockSpec(shape, lambda i: tuple(0 for _ in shape))


def _tc_first(x, W1, degp):
    return pl.pallas_call(
        _tc_first_body,
        grid=(GRID,),
        in_specs=[_row_spec(D), _full((D, D)), _degp_spec()],
        out_specs=[_row_spec(D), _row_spec(D)],
        out_shape=[jax.ShapeDtypeStruct((N, D), jnp.float32)] * 2,
    )(x, W1, degp)


def _tc_mid(sp, aux, degp, b1, W2):
    return pl.pallas_call(
        _tc_mid_body,
        grid=(GRID,),
        in_specs=[pl.BlockSpec((2, BLK, D), lambda i: (0, i, 0)),
                  _row_spec(D), _degp_spec(), _full((1, D)), _full((D, D))],
        out_specs=[_row_spec(D), _row_spec(D)],
        out_shape=[jax.ShapeDtypeStruct((N, D), jnp.float32)] * 2,
    )(sp, aux, degp, b1, W2)


def _tc_head(sp, aux, degp, b2, batch3, Wf1, bf1, Wf2, bf2, Wf3, bf3):
    return pl.pallas_call(
        _tc_head_body,
        grid=(GRID,),
        in_specs=[pl.BlockSpec((2, BLK, D), lambda i: (0, i, 0)),
                  _row_spec(D), _degp_spec(), _full((1, D)),
                  pl.BlockSpec((1, 1, BLK), lambda i: (i, 0, 0)),
                  _full((D, D)), _full((1, D)),
                  _full((D, D)), _full((1, D)),
                  _full((D, D)), _full((1, D))],
        out_specs=pl.BlockSpec((G, D), lambda i: (0, 0)),
        out_shape=jax.ShapeDtypeStruct((G, D), jnp.float32),
        scratch_shapes=[pltpu.VMEM((G, D), jnp.float32),
                        pltpu.VMEM((G, D), jnp.float32)],
    )(sp, aux, degp, b2, batch3, Wf1, bf1, Wf2, bf2, Wf3, bf3)


def _pad2d(w, rows, cols):
    return jnp.pad(w, ((0, rows - w.shape[0]), (0, cols - w.shape[1])))


def kernel(x, edge_index, edge_weight, batch, W1, b1, W2, b2,
           Wf1, bf1, Wf2, bf2, Wf3, bf3):
    E = edge_weight.shape[0]
    pad = NT * EPT - E
    row3 = jnp.pad(edge_index[0], (0, pad)).reshape(NT, NCH, CH)
    col3 = jnp.pad(edge_index[1], (0, pad)).reshape(NT, NCH, CH)
    w3 = jnp.pad(edge_weight, (0, pad)).reshape(NT, NCH, CH)
    batch3 = batch.reshape(GRID, 1, BLK)

    degp = _deg_kernel(col3, w3)

    y1, aux1 = _tc_first(x, W1, degp)
    s1 = _scatter_kernel(y1, row3, col3, w3)
    y2, aux2 = _tc_mid(s1, aux1, degp, b1.reshape(1, D), W2)
    s2 = _scatter_kernel(y2, row3, col3, w3)

    out = _tc_head(s2, aux2, degp, b2.reshape(1, D), batch3,
                   _pad2d(Wf1, D, D),
                   jnp.pad(bf1, (0, D - bf1.shape[0])).reshape(1, D),
                   _pad2d(Wf2, D, D),
                   jnp.pad(bf2, (0, D - bf2.shape[0])).reshape(1, D),
                   _pad2d(Wf3, D, D),
                   jnp.pad(bf3, (0, D - bf3.shape[0])).reshape(1, D))
    return out[:, :10]
